# baseline (speedup 1.0000x reference)
import functools

import jax
import jax.numpy as jnp
from jax import lax
from jax.experimental import pallas as pl
from jax.experimental.pallas import tpu as pltpu
from jax.experimental.pallas import tpu_sc as plsc

_NC = 2
_NS = 16
_NW = _NC * _NS

_VOCAB = 1000000
_EMBED = 16
_B = 16384 * 26
_ROWS_W = _B // _NW
_CHUNK = 128
_GROUP = 13
_GROUP_ROWS = _GROUP * _CHUNK
_NGROUPS = _ROWS_W // _GROUP_ROWS


def _body(idx_hbm, w_hbm, scale_hbm, out_hbm, idx_v, rows_v, scale_v, sem):
    wid = lax.axis_index("s") * _NC + lax.axis_index("c")
    base = wid * _ROWS_W

    pltpu.sync_copy(idx_hbm.at[wid], idx_v)
    pltpu.sync_copy(scale_hbm, scale_v)
    scale = scale_v[...]

    def group(g, carry):
        copies = [
            pltpu.async_copy(
                w_hbm.at[idx_v.at[g * _GROUP + b]],
                rows_v.at[pl.ds(b * _CHUNK, _CHUNK)],
                sem,
            )
            for b in range(_GROUP)
        ]
        for c in copies:
            c.wait()

        def mul(i, carry):
            rows_v[i, :] = rows_v[i, :] * scale
            return carry

        lax.fori_loop(0, _GROUP_ROWS, mul, None)
        pltpu.sync_copy(
            rows_v, out_hbm.at[pl.ds(base + g * _GROUP_ROWS, _GROUP_ROWS)]
        )
        return carry

    lax.fori_loop(0, _NGROUPS, group, None)


@jax.jit
def _gather_scale(idx, weight, scale_vec):
    mesh = plsc.VectorSubcoreMesh(core_axis_name="c", subcore_axis_name="s")
    f = pl.kernel(
        _body,
        out_type=jax.ShapeDtypeStruct((_B, _EMBED), jnp.float32),
        mesh=mesh,
        scratch_types=[
            pltpu.VMEM((_ROWS_W // _CHUNK, _CHUNK), jnp.int32),
            pltpu.VMEM((_GROUP_ROWS, _EMBED), jnp.float32),
            pltpu.VMEM((_EMBED,), jnp.float32),
            pltpu.SemaphoreType.DMA,
        ],
        compiler_params=pltpu.CompilerParams(use_tc_tiling_on_sc=False),
    )
    return f(idx, weight, scale_vec)


def kernel(input, weight, weight_scale):
    idx = input.reshape(_NW, _ROWS_W // _CHUNK, _CHUNK).astype(jnp.int32)
    scale_vec = jnp.broadcast_to(
        weight_scale.astype(jnp.float32), (_EMBED,)
    )
    out = _gather_scale(idx, weight, scale_vec)
    return out.reshape(input.shape + (_EMBED,))
